# baseline (speedup 1.0000x reference)
import functools

import jax
import jax.numpy as jnp
from jax import lax
from jax.experimental import pallas as pl
from jax.experimental.pallas import tpu as pltpu
from jax.experimental.pallas import tpu_sc as plsc

_S = 48
_CHUNK = 128
_LEAD = 4
_LANES = 16


def _make_sc_upsample(n, c, m):
  info = plsc.get_sparse_core_info()
  nw = info.num_cores * info.num_subcores
  rows_per_w = m // nw
  n_chunks = rows_per_w // _CHUNK
  n_groups = n_chunks // _LEAD
  assert m == nw * rows_per_w and rows_per_w == n_chunks * _CHUNK
  assert n_chunks == n_groups * _LEAD and c % _LANES == 0

  mesh = plsc.VectorSubcoreMesh(core_axis_name="c", subcore_axis_name="s")

  @functools.partial(
      pl.kernel,
      out_type=jax.ShapeDtypeStruct((m, c), jnp.float32),
      mesh=mesh,
      compiler_params=pltpu.CompilerParams(needs_layout_passes=False),
      scratch_types=(
          [pltpu.VMEM((rows_per_w,), jnp.int32)]
          + [pltpu.VMEM((_S, c), jnp.float32) for _ in range(_LEAD)]
          + [pltpu.VMEM((_CHUNK, c), jnp.float32) for _ in range(_LEAD)]
          + [pltpu.SemaphoreType.DMA for _ in range(2 * _LEAD)]
      ),
  )
  def upsample_kernel(data_hbm, idx_hbm, out_hbm, idx_v, *refs):
    sbufs = refs[:_LEAD]
    obufs = refs[_LEAD : 2 * _LEAD]
    ssem = refs[2 * _LEAD : 3 * _LEAD]
    osem = refs[3 * _LEAD :]
    wid = lax.axis_index("s") * info.num_cores + lax.axis_index("c")
    base = wid * rows_per_w

    pltpu.sync_copy(idx_hbm.at[pl.ds(base, rows_per_w)], idx_v)

    def shift_body(i, carry):
      sl = pl.ds(i * _LANES, _LANES)
      idx_v[sl] = lax.shift_right_logical(idx_v[sl], 3)
      return carry

    lax.fori_loop(0, rows_per_w // _LANES, shift_body, 0)

    iota16 = lax.iota(jnp.int32, _LANES)

    def chunk_lo(ch):
      v = idx_v[pl.ds(ch * _CHUNK, _LANES)]
      return jnp.bitwise_and(v[0], -8)

    def chunk_hi(ch):
      v = idx_v[pl.ds(ch * _CHUNK + _CHUNK - _LANES, _LANES)]
      return v[_LANES - 1]

    def win_start(lo, w):
      return pl.multiple_of(jnp.minimum(lo + w * _S, n - _S), 8)

    def issue_span(ch, b):
      st = win_start(chunk_lo(ch), 0)
      pltpu.async_copy(data_hbm.at[pl.ds(st, _S)], sbufs[b], ssem[b])

    def drain_span(ch, b):
      st = win_start(chunk_lo(ch), 0)
      pltpu.make_async_copy(
          data_hbm.at[pl.ds(st, _S)], sbufs[b], ssem[b]
      ).wait()

    def issue_write(ch, b):
      pltpu.async_copy(
          obufs[b], out_hbm.at[pl.ds(base + ch * _CHUNK, _CHUNK)], osem[b]
      )

    def drain_write(ch, b):
      pltpu.make_async_copy(
          obufs[b], out_hbm.at[pl.ds(base + ch * _CHUNK, _CHUNK)], osem[b]
      ).wait()

    def expand_window(ch, b, st):
      stv = jnp.full((_LANES,), st, jnp.int32)

      def rg_body(rg, carry):
        p16 = idx_v[pl.ds(ch * _CHUNK + rg * _LANES, _LANES)]
        off = p16 - stv
        mask = jnp.logical_and(off >= 0, off < _S)
        offc = jnp.clip(off, 0, _S - 1)
        rows = iota16 + rg * _LANES
        for cb in range(c // _LANES):
          for l in range(_LANES):
            colv = jnp.full((_LANES,), cb * _LANES + l, jnp.int32)
            v = plsc.load_gather(sbufs[b], [offc, colv])
            plsc.store_scatter(obufs[b], [rows, colv], v, mask=mask)
        return carry

      lax.fori_loop(0, _CHUNK // _LANES, rg_body, 0)

    def expand_chunk(ch, b):
      lo = chunk_lo(ch)
      hi = chunk_hi(ch)
      expand_window(ch, b, win_start(lo, 0))
      nwin = (hi - lo) // _S + 1

      def wbody(w, carry):
        st = win_start(lo, w)
        pltpu.sync_copy(data_hbm.at[pl.ds(st, _S)], sbufs[b])
        expand_window(ch, b, st)
        return carry

      lax.fori_loop(1, nwin, wbody, 0)

    for b in range(_LEAD):
      issue_span(b, b)

    def group_body(g, carry):
      for j in range(_LEAD):
        ch = g * _LEAD + j

        @pl.when(g > 0)
        def _():
          drain_write(ch - _LEAD, j)

        drain_span(ch, j)
        expand_chunk(ch, j)
        issue_write(ch, j)

        @pl.when(g < n_groups - 1)
        def _():
          issue_span(ch + _LEAD, j)

      return carry

    lax.fori_loop(0, n_groups, group_body, 0)

    for j in range(_LEAD):
      drain_write(n_chunks - _LEAD + j, j)

    return None

  return upsample_kernel


def kernel(data, child_idx, depth):
  n, c = data.shape
  (m,) = child_idx.shape
  return _make_sc_upsample(n, c, m)(data, child_idx)


# speedup vs baseline: 1.4203x; 1.4203x over previous
import functools

import jax
import jax.numpy as jnp
from jax import lax
from jax.experimental import pallas as pl
from jax.experimental.pallas import tpu as pltpu
from jax.experimental.pallas import tpu_sc as plsc

_S = 48
_CHUNK = 128
_LEAD = 4
_LANES = 16


def _make_sc_upsample(n, c, m):
  info = plsc.get_sparse_core_info()
  nw = info.num_cores * info.num_subcores
  rows_per_w = m // nw
  n_chunks = rows_per_w // _CHUNK
  n_groups = n_chunks // _LEAD
  assert m == nw * rows_per_w and rows_per_w == n_chunks * _CHUNK
  assert n_chunks == n_groups * _LEAD and c % _LANES == 0

  mesh = plsc.VectorSubcoreMesh(core_axis_name="c", subcore_axis_name="s")

  @functools.partial(
      pl.kernel,
      out_type=jax.ShapeDtypeStruct((m, c), jnp.float32),
      mesh=mesh,
      compiler_params=pltpu.CompilerParams(needs_layout_passes=False),
      scratch_types=(
          [pltpu.VMEM((rows_per_w,), jnp.int32)]
          + [pltpu.VMEM((_S, c), jnp.float32) for _ in range(_LEAD)]
          + [pltpu.VMEM((_CHUNK, c), jnp.float32) for _ in range(_LEAD)]
          + [pltpu.SemaphoreType.DMA for _ in range(2 * _LEAD)]
      ),
  )
  def upsample_kernel(data_hbm, idx_hbm, out_hbm, idx_v, *refs):
    sbufs = refs[:_LEAD]
    obufs = refs[_LEAD : 2 * _LEAD]
    ssem = refs[2 * _LEAD : 3 * _LEAD]
    osem = refs[3 * _LEAD :]
    wid = lax.axis_index("s") * info.num_cores + lax.axis_index("c")
    base = wid * rows_per_w

    pltpu.sync_copy(idx_hbm.at[pl.ds(base, rows_per_w)], idx_v)

    def shift_body(i, carry):
      sl = pl.ds(i * _LANES, _LANES)
      idx_v[sl] = lax.shift_right_logical(idx_v[sl], 3)
      return carry

    lax.fori_loop(0, rows_per_w // _LANES, shift_body, 0)

    iota16 = lax.iota(jnp.int32, _LANES)

    def chunk_lo(ch):
      v = idx_v[pl.ds(ch * _CHUNK, _LANES)]
      return jnp.bitwise_and(v[0], -8)

    def chunk_hi(ch):
      v = idx_v[pl.ds(ch * _CHUNK + _CHUNK - _LANES, _LANES)]
      return v[_LANES - 1]

    def win_start(lo, w):
      return pl.multiple_of(jnp.minimum(lo + w * _S, n - _S), 8)

    def issue_span(ch, b):
      st = win_start(chunk_lo(ch), 0)
      pltpu.async_copy(data_hbm.at[pl.ds(st, _S)], sbufs[b], ssem[b])

    def drain_span(ch, b):
      st = win_start(chunk_lo(ch), 0)
      pltpu.make_async_copy(
          data_hbm.at[pl.ds(st, _S)], sbufs[b], ssem[b]
      ).wait()

    def issue_write(ch, b):
      pltpu.async_copy(
          obufs[b], out_hbm.at[pl.ds(base + ch * _CHUNK, _CHUNK)], osem[b]
      )

    def drain_write(ch, b):
      pltpu.make_async_copy(
          obufs[b], out_hbm.at[pl.ds(base + ch * _CHUNK, _CHUNK)], osem[b]
      ).wait()

    def expand_window(ch, b, st):
      stv = jnp.full((_LANES,), st, jnp.int32)

      def rg_body(rg, carry):
        p16 = idx_v[pl.ds(ch * _CHUNK + rg * _LANES, _LANES)]
        off = p16 - stv
        mask = jnp.logical_and(off >= 0, off < _S)
        offc = jnp.clip(off, 0, _S - 1)
        rows = iota16 + rg * _LANES
        for cb in range(c // _LANES):
          colvs = [
              jnp.full((_LANES,), cb * _LANES + l, jnp.int32)
              for l in range(_LANES)
          ]
          vals = [
              plsc.load_gather(sbufs[b], [offc, colvs[l]])
              for l in range(_LANES)
          ]
          for l in range(_LANES):
            plsc.store_scatter(obufs[b], [rows, colvs[l]], vals[l], mask=mask)
        return carry

      lax.fori_loop(0, _CHUNK // _LANES, rg_body, 0)

    def expand_chunk(ch, b):
      lo = chunk_lo(ch)
      hi = chunk_hi(ch)
      expand_window(ch, b, win_start(lo, 0))
      nwin = (hi - lo) // _S + 1

      def wbody(w, carry):
        st = win_start(lo, w)
        pltpu.sync_copy(data_hbm.at[pl.ds(st, _S)], sbufs[b])
        expand_window(ch, b, st)
        return carry

      lax.fori_loop(1, nwin, wbody, 0)

    for b in range(_LEAD):
      issue_span(b, b)

    def group_body(g, carry):
      for j in range(_LEAD):
        ch = g * _LEAD + j

        @pl.when(g > 0)
        def _():
          drain_write(ch - _LEAD, j)

        drain_span(ch, j)
        expand_chunk(ch, j)
        issue_write(ch, j)

        @pl.when(g < n_groups - 1)
        def _():
          issue_span(ch + _LEAD, j)

      return carry

    lax.fori_loop(0, n_groups, group_body, 0)

    for j in range(_LEAD):
      drain_write(n_chunks - _LEAD + j, j)

    return None

  return upsample_kernel


def kernel(data, child_idx, depth):
  n, c = data.shape
  (m,) = child_idx.shape
  return _make_sc_upsample(n, c, m)(data, child_idx)


# speedup vs baseline: 9.4919x; 6.6828x over previous
import functools

import jax
import jax.numpy as jnp
from jax import lax
from jax.experimental import pallas as pl
from jax.experimental.pallas import tpu as pltpu
from jax.experimental.pallas import tpu_sc as plsc

_NBUF = 8
_CHUNK = 64
_LANES = 16


def _make_sc_gather(n, c, m):
  info = plsc.get_sparse_core_info()
  nw = info.num_cores * info.num_subcores
  rows_per_w = m // nw
  n_chunks = rows_per_w // _CHUNK
  n_groups = n_chunks // _NBUF
  assert m == nw * rows_per_w and rows_per_w == n_chunks * _CHUNK
  assert n_chunks == n_groups * _NBUF

  mesh = plsc.VectorSubcoreMesh(core_axis_name="c", subcore_axis_name="s")

  @functools.partial(
      pl.kernel,
      out_type=jax.ShapeDtypeStruct((m, c), jnp.float32),
      mesh=mesh,
      scratch_types=(
          [pltpu.VMEM((rows_per_w,), jnp.int32)]
          + [pltpu.VMEM((_CHUNK, c), jnp.float32) for _ in range(_NBUF)]
          + [pltpu.SemaphoreType.DMA for _ in range(_NBUF)]
      ),
  )
  def gather_kernel(data_hbm, idx_hbm, out_hbm, idx_v, *bufs_sems):
    bufs = bufs_sems[:_NBUF]
    sems = bufs_sems[_NBUF:]
    wid = lax.axis_index("s") * info.num_cores + lax.axis_index("c")
    base = wid * rows_per_w

    pltpu.sync_copy(idx_hbm.at[pl.ds(base, rows_per_w)], idx_v)

    def shift_body(i, carry):
      sl = pl.ds(i * _LANES, _LANES)
      idx_v[sl] = lax.shift_right_logical(idx_v[sl], 3)
      return carry

    lax.fori_loop(0, rows_per_w // _LANES, shift_body, 0)

    def start(chunk, b):
      pltpu.async_copy(
          data_hbm.at[idx_v.at[pl.ds(chunk * _CHUNK, _CHUNK)]],
          bufs[b],
          sems[b],
      )

    def drain(chunk, b):
      pltpu.make_async_copy(
          data_hbm.at[idx_v.at[pl.ds(chunk * _CHUNK, _CHUNK)]],
          bufs[b],
          sems[b],
      ).wait()

    for b in range(_NBUF):
      start(b, b)

    def group_body(g, carry):
      for b in range(_NBUF):
        chunk = g * _NBUF + b
        drain(chunk, b)
        pltpu.sync_copy(
            bufs[b], out_hbm.at[pl.ds(base + chunk * _CHUNK, _CHUNK)]
        )
        start(chunk + _NBUF, b)
      return carry

    lax.fori_loop(0, n_groups - 1, group_body, 0)

    for b in range(_NBUF):
      chunk = (n_groups - 1) * _NBUF + b
      drain(chunk, b)
      pltpu.sync_copy(
          bufs[b], out_hbm.at[pl.ds(base + chunk * _CHUNK, _CHUNK)]
      )

  return gather_kernel


def kernel(data, child_idx, depth):
  n, c = data.shape
  (m,) = child_idx.shape
  return _make_sc_gather(n, c, m)(data, child_idx)
